# baseline (speedup 1.0000x reference)
import functools

import jax
import jax.numpy as jnp
from jax import lax
from jax.experimental import pallas as pl
from jax.experimental.pallas import tpu as pltpu
from jax.experimental.pallas import tpu_sc as plsc

VOCAB = 1000000
DIM = 64
MAXLEN = 200
BATCH = 1024

NTQ = 4
TQ = MAXLEN // NTQ
NBB = 8
BB = BATCH // NBB


def _emb_kernel(xT_hbm, tok_hbm, posT_hbm, out_hbm,
                xstage_v, idx_v, rows_v, out_v, posT_v, sem):
    c = lax.axis_index("c")
    s = lax.axis_index("s")
    wid = s * 2 + c
    t0 = (wid // NBB) * TQ
    b0 = (wid % NBB) * BB

    pltpu.sync_copy(posT_hbm, posT_v)
    pltpu.sync_copy(xT_hbm.at[:, pl.ds(b0, BB)], xstage_v)

    def conv(j, _):
        for m in range(BB // 16):
            sl = pl.ds(m * 16, 16)
            idx_v[j, sl] = xstage_v[t0 + j, sl].astype(jnp.int32)
        return 0

    lax.fori_loop(0, TQ, conv, 0)

    iota16 = lax.broadcasted_iota(jnp.int32, (16,), 0)
    iotas = [iota16 + m * 16 for m in range(BB // 16)]

    def fetch(j, buf):
        return pltpu.async_copy(
            tok_hbm.at[idx_v.at[j]], rows_v.at[pl.ds(buf * BB, BB)], sem)

    fetch(0, 0)

    def step(j, _):
        t = t0 + j
        buf = lax.rem(j, 2)
        rv = rows_v.at[pl.ds(buf * BB, BB)]
        pltpu.make_async_copy(tok_hbm.at[idx_v.at[j]], rv, sem).wait()

        @pl.when(j + 1 < TQ)
        def _():
            fetch(j + 1, 1 - buf)

        tsplat = jnp.full((16,), t, jnp.int32)
        for dd in range(DIM // 16):
            pvs = plsc.load_gather(posT_v, [iotas[dd], tsplat])
            ps = [pvs[l] for l in range(16)]
            for m in range(BB // 16):
                msl = pl.ds(m * 16, 16)
                for l in range(16):
                    d = dd * 16 + l
                    g = plsc.load_gather(
                        rv, [iotas[m], jnp.full((16,), d, jnp.int32)])
                    out_v[d, msl] = g + ps[l]

        pltpu.sync_copy(out_v, out_hbm.at[t, :, pl.ds(b0, BB)])
        return 0

    lax.fori_loop(0, TQ, step, 0)


def kernel(x, token_table, pos_table):
    xT = x.astype(jnp.float32).T
    posT = pos_table.T
    mesh = plsc.VectorSubcoreMesh(core_axis_name="c", subcore_axis_name="s")
    run = functools.partial(
        pl.kernel,
        mesh=mesh,
        out_type=jax.ShapeDtypeStruct((MAXLEN, DIM, BATCH), jnp.float32),
        scratch_types=[
            pltpu.VMEM((MAXLEN, BB), jnp.float32),
            pltpu.VMEM((TQ, BB), jnp.int32),
            pltpu.VMEM((2 * BB, DIM), jnp.float32),
            pltpu.VMEM((DIM, BB), jnp.float32),
            pltpu.VMEM((DIM, MAXLEN), jnp.float32),
            pltpu.SemaphoreType.DMA,
        ],
        compiler_params=pltpu.CompilerParams(
            use_tc_tiling_on_sc=False, needs_layout_passes=False),
    )(_emb_kernel)
    oT = run(xT, token_table, posT)
    return oT.transpose(2, 0, 1)


# speedup vs baseline: 1.0289x; 1.0289x over previous
import functools

import jax
import jax.numpy as jnp
from jax import lax
from jax.experimental import pallas as pl
from jax.experimental.pallas import tpu as pltpu
from jax.experimental.pallas import tpu_sc as plsc

VOCAB = 1000000
DIM = 64
MAXLEN = 200
BATCH = 1024

NTQ = 4
TQ = MAXLEN // NTQ
NBB = 8
BB = BATCH // NBB
TSTAGE = 56
PITCH = DIM + 1


def _emb_kernel(xT_hbm, tok_hbm, posT_hbm, out_hbm,
                xstage_v, idx_v, hsel_v, rows_v, st_v, out_v, pos_v, posT_v,
                sem):
    c = lax.axis_index("c")
    s = lax.axis_index("s")
    wid = s * 2 + c
    t0 = (wid // NBB) * TQ
    t0a = (t0 // 8) * 8
    toff = t0 - t0a
    b0 = (wid % NBB) * BB

    pltpu.sync_copy(posT_hbm, posT_v)
    pltpu.sync_copy(xT_hbm.at[pl.ds(t0a, TSTAGE), pl.ds(b0, BB)], xstage_v)

    iota16 = lax.broadcasted_iota(jnp.int32, (16,), 0)

    def pos_tr(t, _):
        tsplat = jnp.full((16,), t, jnp.int32)
        for dd in range(DIM // 16):
            pos_v[t, pl.ds(dd * 16, 16)] = plsc.load_gather(
                posT_v, [iota16 + dd * 16, tsplat])
        return 0

    lax.fori_loop(0, MAXLEN, pos_tr, 0)

    def conv(j, _):
        for m in range(BB // 16):
            sl = pl.ds(m * 16, 16)
            tok = xstage_v[toff + j, sl].astype(jnp.int32)
            idx_v[j, sl] = lax.shift_right_logical(tok, 1)
            hsel_v[j, sl] = (tok & 1) * DIM
        return 0

    lax.fori_loop(0, TQ, conv, 0)

    bases = [(iota16 + m * 16) * PITCH for m in range(BB // 16)]

    def fetch(j, buf):
        return pltpu.async_copy(
            tok_hbm.at[idx_v.at[j]], rows_v.at[pl.ds(buf * BB, BB)], sem)

    fetch(0, 0)

    def step(j, _):
        t = t0 + j
        buf = lax.rem(j, 2)
        rbase = buf * BB
        pltpu.make_async_copy(
            tok_hbm.at[idx_v.at[j]], rows_v.at[pl.ds(rbase, BB)], sem).wait()

        @pl.when(j + 1 < TQ)
        def _():
            fetch(j + 1, 1 - buf)

        pos4 = [pos_v[t, pl.ds(dd * 16, 16)] for dd in range(DIM // 16)]

        for g in range(BB // 16):
            hv = hsel_v[j, pl.ds(g * 16, 16)]
            for k in range(16):
                r = g * 16 + k
                h = hv[k]
                for dd in range(DIM // 16):
                    st_v[pl.ds(r * PITCH + dd * 16, 16)] = (
                        rows_v[rbase + r, pl.ds(h + dd * 16, 16)] + pos4[dd])

        for dd in range(DIM // 16):
            for l in range(16):
                d = dd * 16 + l
                for m in range(BB // 16):
                    out_v[d, pl.ds(m * 16, 16)] = plsc.load_gather(
                        st_v, [bases[m] + d])

        pltpu.sync_copy(out_v, out_hbm.at[t, :, pl.ds(b0, BB)])
        return 0

    lax.fori_loop(0, TQ, step, 0)


def kernel(x, token_table, pos_table):
    xT = x.astype(jnp.float32).T
    posT = pos_table.T
    tok2 = token_table.reshape(VOCAB // 2, 2 * DIM)
    mesh = plsc.VectorSubcoreMesh(core_axis_name="c", subcore_axis_name="s")
    run = functools.partial(
        pl.kernel,
        mesh=mesh,
        out_type=jax.ShapeDtypeStruct((MAXLEN, DIM, BATCH), jnp.float32),
        scratch_types=[
            pltpu.VMEM((TSTAGE, BB), jnp.float32),
            pltpu.VMEM((TQ, BB), jnp.int32),
            pltpu.VMEM((TQ, BB), jnp.int32),
            pltpu.VMEM((2 * BB, 2 * DIM), jnp.float32),
            pltpu.VMEM((BB * PITCH,), jnp.float32),
            pltpu.VMEM((DIM, BB), jnp.float32),
            pltpu.VMEM((MAXLEN, DIM), jnp.float32),
            pltpu.VMEM((DIM, MAXLEN), jnp.float32),
            pltpu.SemaphoreType.DMA,
        ],
        compiler_params=pltpu.CompilerParams(needs_layout_passes=False),
    )(_emb_kernel)
    oT = run(xT, tok2, posT)
    return oT.transpose(2, 0, 1)


# speedup vs baseline: 1.2403x; 1.2055x over previous
import functools

import jax
import jax.numpy as jnp
from jax import lax
from jax.experimental import pallas as pl
from jax.experimental.pallas import tpu as pltpu
from jax.experimental.pallas import tpu_sc as plsc

VOCAB = 1000000
DIM = 64
MAXLEN = 200
BATCH = 1024

TOKENS = BATCH * MAXLEN
NW = 32
PER_W = TOKENS // NW
CHUNK = 128
GATHERS = CHUNK // 128
STEPS = PER_W // CHUNK


def _emb_kernel(x_hbm, tok_hbm, pos_hbm, out_hbm,
                xraw_v, idx2_v, rows_v, out_v, pos_v,
                sg0, sg1, so0, so1):
    c = lax.axis_index("c")
    s = lax.axis_index("s")
    wid = s * 2 + c

    pltpu.sync_copy(pos_hbm, pos_v)

    sgs = (sg0, sg1)
    sos = (so0, so1)

    def load_idx(j, b):
        pltpu.sync_copy(x_hbm.at[wid * STEPS + j], xraw_v.at[b])
        for g in range(GATHERS):
            for m in range(128 // 16):
                sl = pl.ds(m * 16, 16)
                idx2_v[b, g, sl] = lax.shift_right_logical(xraw_v[b, g, sl], 1)

    def fire_gathers(j, b):
        for g in range(GATHERS):
            pltpu.async_copy(
                tok_hbm.at[idx2_v.at[b, g]],
                rows_v.at[pl.ds((b * GATHERS + g) * 128, 128)],
                sgs[b])

    def wait_gathers(j, b):
        for g in range(GATHERS):
            pltpu.make_async_copy(
                tok_hbm.at[idx2_v.at[b, g]],
                rows_v.at[pl.ds((b * GATHERS + g) * 128, 128)],
                sgs[b]).wait()

    def out_dma_refs(j, b):
        base = wid * PER_W + j * CHUNK
        return out_v.at[pl.ds(b * CHUNK, CHUNK)], out_hbm.at[pl.ds(base, CHUNK)]

    load_idx(0, 0)
    fire_gathers(0, 0)
    load_idx(1, 1)
    fire_gathers(1, 1)

    def step_body(j, b):
        off = lax.rem(j * CHUNK, MAXLEN)
        wait_gathers(j, b)

        @pl.when(j >= 2)
        def _():
            src, dst = out_dma_refs(j - 2, b)
            pltpu.make_async_copy(src, dst, sos[b]).wait()

        def add_pos(g16, _):
            for blk in range(GATHERS):
                hv = (xraw_v[b, blk, pl.ds(g16 * 16, 16)] & 1) * DIM
                for l in range(16):
                    row = blk * 128 + g16 * 16 + l
                    h = hv[l]
                    p = lax.rem(off + row, MAXLEN)
                    for dd in range(DIM // 16):
                        sl = pl.ds(dd * 16, 16)
                        out_v[b * CHUNK + row, sl] = (
                            rows_v[(b * GATHERS + blk) * 128 + g16 * 16 + l,
                                   pl.ds(h + dd * 16, 16)]
                            + pos_v[p, sl])
            return 0

        lax.fori_loop(0, 128 // 16, add_pos, 0)

        @pl.when(j + 2 < STEPS)
        def _():
            load_idx(j + 2, b)
            fire_gathers(j + 2, b)

        src, dst = out_dma_refs(j, b)
        pltpu.async_copy(src, dst, sos[b])

    def step(j, _):
        for b in range(2):
            @pl.when(lax.rem(j, 2) == b)
            def _(b=b):
                step_body(j, b)
        return 0

    lax.fori_loop(0, STEPS, step, 0)

    for jj in (STEPS - 2, STEPS - 1):
        src, dst = out_dma_refs(jj, jj % 2)
        pltpu.make_async_copy(src, dst, sos[jj % 2]).wait()


def kernel(x, token_table, pos_table):
    xf = x.reshape(NW * STEPS, GATHERS, 128).astype(jnp.int32)
    tok2 = token_table.reshape(VOCAB // 2, 2 * DIM)
    mesh = plsc.VectorSubcoreMesh(core_axis_name="c", subcore_axis_name="s")
    run = functools.partial(
        pl.kernel,
        mesh=mesh,
        out_type=jax.ShapeDtypeStruct((TOKENS, DIM), jnp.float32),
        scratch_types=[
            pltpu.VMEM((2, GATHERS, 128), jnp.int32),
            pltpu.VMEM((2, GATHERS, 128), jnp.int32),
            pltpu.VMEM((2 * CHUNK, 2 * DIM), jnp.float32),
            pltpu.VMEM((2 * CHUNK, DIM), jnp.float32),
            pltpu.VMEM((MAXLEN, DIM), jnp.float32),
            pltpu.SemaphoreType.DMA,
            pltpu.SemaphoreType.DMA,
            pltpu.SemaphoreType.DMA,
            pltpu.SemaphoreType.DMA,
        ],
    )(_emb_kernel)
    out = run(xf, tok2, pos_table)
    return out.reshape(BATCH, MAXLEN, DIM)
